# baseline (speedup 1.0000x reference)
import functools
import math

import jax
import jax.numpy as jnp
import numpy as np
from jax import lax
from jax.experimental import pallas as pl
from jax.experimental.pallas import tpu as pltpu
from jax.experimental.pallas import tpu_sc as plsc

NUM_EMBED = 1000000
EMBED_DIM = 64
SEQ_LEN = 200
BATCH = 4096
BL = BATCH * SEQ_LEN

NC, NS = 2, 16
NW = NC * NS
CHUNK = 128
NUNIT = BL // CHUNK
PER_W = NUNIT // NW
BBLK = BATCH // CHUNK
LANES = 16
DT = EMBED_DIM // 8
DG = EMBED_DIM // LANES
NBUF = 5
LOOKAHEAD = NBUF - 1


def _pe() -> np.ndarray:
    position = np.arange(0, SEQ_LEN, dtype=np.float32)[:, None]
    div_term = np.exp(
        np.arange(0, EMBED_DIM, 2, dtype=np.float32)
        * (-math.log(10000.0) / EMBED_DIM)
    )
    pe = np.zeros((SEQ_LEN, EMBED_DIM), dtype=np.float32)
    pe[:, 0::2] = np.sin(position * div_term)
    pe[:, 1::2] = np.cos(position * div_term)
    return pe


@functools.partial(
    pl.kernel,
    out_type=jax.ShapeDtypeStruct((SEQ_LEN, DT, BBLK, 8 * CHUNK), jnp.float32),
    mesh=plsc.VectorSubcoreMesh(
        core_axis_name="c", subcore_axis_name="s", num_cores=NC, num_subcores=NS
    ),
    scratch_types=[
        pltpu.VMEM((PER_W * CHUNK,), jnp.int32),
        pltpu.VMEM((SEQ_LEN, EMBED_DIM), jnp.float32),
    ]
    + [pltpu.VMEM((CHUNK, EMBED_DIM), jnp.float32) for _ in range(NBUF)]
    + [pltpu.VMEM((DT, 8 * CHUNK), jnp.float32) for _ in range(NBUF)]
    + [pltpu.SemaphoreType.DMA for _ in range(2 * NBUF)],
    compiler_params=pltpu.CompilerParams(
        use_tc_tiling_on_sc=False, needs_layout_passes=False
    ),
)
def _sc_embed(xt_hbm, table_hbm, pe_hbm, out_hbm, idx_v, pe_v, *bufs):
    rows = bufs[:NBUF]
    outs = bufs[NBUF : 2 * NBUF]
    gsems = bufs[2 * NBUF : 3 * NBUF]
    osems = bufs[3 * NBUF :]

    wid = lax.axis_index("s") * NC + lax.axis_index("c")
    ubase = wid * PER_W
    pltpu.sync_copy(xt_hbm.at[pl.ds(ubase * CHUNK, PER_W * CHUNK)], idx_v)
    pltpu.sync_copy(pe_hbm, pe_v)
    jvec = lax.iota(jnp.int32, LANES)

    @plsc.parallel_loop(0, PER_W * CHUNK // LANES, 1, unroll=8)
    def _dbl(i):
        sl = pl.ds(i * LANES, LANES)
        idx_v[sl] = idx_v[sl] * 2
    rks = [(jvec + k) % LANES for k in range(LANES)]
    rdivs = [rk // 8 for rk in rks]
    cvecs = [(rk % 8) * CHUNK + jvec for rk in rks]

    def issue_gather(u, b):
        pltpu.async_copy(
            table_hbm.at[idx_v.at[pl.ds(u * CHUNK, CHUNK)]], rows[b], gsems[b]
        )

    def wait_gather(b):
        pltpu.make_async_copy(
            table_hbm.at[pl.ds(0, CHUNK)], rows[b], gsems[b]
        ).wait()

    def drain_out(b):
        pltpu.make_async_copy(
            outs[b], out_hbm.at[0, :, 0], osems[b]
        ).wait()

    for b in range(LOOKAHEAD):
        issue_gather(jnp.int32(b), b)

    def group_body(g4, carry):
        for b in range(NBUF):
            u = g4 * NBUF + b
            g = ubase + u
            l = g // BBLK
            bt = g % BBLK
            ob = b
            wait_gather(b)

            @pl.when(u >= NBUF)
            def _():
                drain_out(ob)

            @plsc.parallel_loop(0, (CHUNK // LANES) * DG, 1, unroll=8)
            def _tp(i):
                bg = i // DG
                dg = i % DG
                bvec = bg * LANES + jvec
                pe16 = pe_v[l, pl.ds(dg * LANES, LANES)]
                for k in range(LANES):
                    dvec = rks[k] + dg * LANES
                    vec = plsc.load_gather(rows[b], [bvec, dvec])
                    vec = vec + jnp.take(pe16, rks[k])
                    plsc.store_scatter(
                        outs[ob],
                        [rdivs[k] + 2 * dg, cvecs[k] + bg * LANES],
                        vec,
                    )

            pltpu.async_copy(outs[ob], out_hbm.at[l, :, bt], osems[ob])

            nb = (b + LOOKAHEAD) % NBUF
            nxt = u + LOOKAHEAD

            @pl.when(nxt < PER_W)
            def _():
                issue_gather(nxt, nb)

        return carry

    lax.fori_loop(0, PER_W // NBUF, group_body, 0)
    for b in range(NBUF):
        drain_out(b)


def kernel(x, table):
    xt = jnp.transpose(x).reshape(-1)
    pe = jnp.asarray(_pe())
    tbig = jnp.pad(table, ((0, 0), (0, EMBED_DIM))).reshape(-1, EMBED_DIM)
    out4 = _sc_embed(xt, tbig, pe)
    out5 = out4.reshape(SEQ_LEN, DT, BBLK, 8, CHUNK)
    out_ldb = out5.transpose(0, 1, 3, 2, 4).reshape(SEQ_LEN, EMBED_DIM, BATCH)
    return out_ldb.transpose(2, 0, 1)


# speedup vs baseline: 1.0058x; 1.0058x over previous
import functools
import math

import jax
import jax.numpy as jnp
import numpy as np
from jax import lax
from jax.experimental import pallas as pl
from jax.experimental.pallas import tpu as pltpu
from jax.experimental.pallas import tpu_sc as plsc

NUM_EMBED = 1000000
EMBED_DIM = 64
SEQ_LEN = 200
BATCH = 4096
BL = BATCH * SEQ_LEN

NC, NS = 2, 16
NW = NC * NS
CHUNK = 128
NUNIT = BL // CHUNK
PER_W = NUNIT // NW
BBLK = BATCH // CHUNK
LANES = 16
DT = EMBED_DIM // 8
DG = EMBED_DIM // LANES
NBUF = 4
LOOKAHEAD = NBUF - 1


def _pe() -> np.ndarray:
    position = np.arange(0, SEQ_LEN, dtype=np.float32)[:, None]
    div_term = np.exp(
        np.arange(0, EMBED_DIM, 2, dtype=np.float32)
        * (-math.log(10000.0) / EMBED_DIM)
    )
    pe = np.zeros((SEQ_LEN, EMBED_DIM), dtype=np.float32)
    pe[:, 0::2] = np.sin(position * div_term)
    pe[:, 1::2] = np.cos(position * div_term)
    return pe


@functools.partial(
    pl.kernel,
    out_type=jax.ShapeDtypeStruct((SEQ_LEN, DT, BBLK, 8 * CHUNK), jnp.float32),
    mesh=plsc.VectorSubcoreMesh(
        core_axis_name="c", subcore_axis_name="s", num_cores=NC, num_subcores=NS
    ),
    scratch_types=[
        pltpu.VMEM((PER_W * CHUNK,), jnp.int32),
        pltpu.VMEM((SEQ_LEN, EMBED_DIM), jnp.float32),
    ]
    + [pltpu.VMEM((CHUNK, EMBED_DIM), jnp.float32) for _ in range(NBUF)]
    + [pltpu.VMEM((DT, 8 * CHUNK), jnp.float32) for _ in range(2)]
    + [pltpu.SemaphoreType.DMA for _ in range(NBUF + 2)],
    compiler_params=pltpu.CompilerParams(
        use_tc_tiling_on_sc=False, needs_layout_passes=False
    ),
)
def _sc_embed(xt_hbm, table_hbm, pe_hbm, out_hbm, idx_v, pe_v, *bufs):
    rows = bufs[:NBUF]
    outs = bufs[NBUF : NBUF + 2]
    gsems = bufs[NBUF + 2 : 2 * NBUF + 2]
    osems = bufs[2 * NBUF + 2 :]

    wid = lax.axis_index("s") * NC + lax.axis_index("c")
    ubase = wid * PER_W
    pltpu.sync_copy(xt_hbm.at[pl.ds(ubase * CHUNK, PER_W * CHUNK)], idx_v)
    pltpu.sync_copy(pe_hbm, pe_v)
    jvec = lax.iota(jnp.int32, LANES)

    @plsc.parallel_loop(0, PER_W * CHUNK // LANES, 1, unroll=8)
    def _dbl(i):
        sl = pl.ds(i * LANES, LANES)
        idx_v[sl] = idx_v[sl] * 2
    rks = [(jvec + k) % LANES for k in range(LANES)]
    rdivs = [rk // 8 for rk in rks]
    cvecs = [(rk % 8) * CHUNK + jvec for rk in rks]

    def issue_gather(u, b):
        pltpu.async_copy(
            table_hbm.at[idx_v.at[pl.ds(u * CHUNK, CHUNK)]], rows[b], gsems[b]
        )

    def wait_gather(b):
        pltpu.make_async_copy(
            table_hbm.at[pl.ds(0, CHUNK)], rows[b], gsems[b]
        ).wait()

    def drain_out(b):
        pltpu.make_async_copy(
            outs[b], out_hbm.at[0, :, 0], osems[b]
        ).wait()

    for b in range(LOOKAHEAD):
        issue_gather(jnp.int32(b), b)

    def group_body(g4, carry):
        for b in range(NBUF):
            u = g4 * NBUF + b
            g = ubase + u
            l = g // BBLK
            bt = g % BBLK
            ob = b % 2
            wait_gather(b)

            @pl.when(u >= 2)
            def _():
                drain_out(ob)

            @plsc.parallel_loop(0, (CHUNK // LANES) * DG, 1, unroll=8)
            def _tp(i):
                bg = i // DG
                dg = i % DG
                bvec = bg * LANES + jvec
                pe16 = pe_v[l, pl.ds(dg * LANES, LANES)]
                for k in range(LANES):
                    dvec = rks[k] + dg * LANES
                    vec = plsc.load_gather(rows[b], [bvec, dvec])
                    vec = vec + jnp.take(pe16, rks[k])
                    plsc.store_scatter(
                        outs[ob],
                        [rdivs[k] + 2 * dg, cvecs[k] + bg * LANES],
                        vec,
                    )

            pltpu.async_copy(outs[ob], out_hbm.at[l, :, bt], osems[ob])

            nb = (b + LOOKAHEAD) % NBUF
            nxt = u + LOOKAHEAD

            @pl.when(nxt < PER_W)
            def _():
                issue_gather(nxt, nb)

        return carry

    lax.fori_loop(0, PER_W // NBUF, group_body, 0)
    for b in range(2):
        drain_out(b)


def kernel(x, table):
    xt = jnp.transpose(x).reshape(-1)
    pe = jnp.asarray(_pe())
    tbig = jnp.pad(table, ((0, 0), (0, EMBED_DIM))).reshape(-1, EMBED_DIM)
    out4 = _sc_embed(xt, tbig, pe)
    out5 = out4.reshape(SEQ_LEN, DT, BBLK, 8, CHUNK)
    out_ldb = out5.transpose(0, 1, 3, 2, 4).reshape(SEQ_LEN, EMBED_DIM, BATCH)
    return out_ldb.transpose(2, 0, 1)
